# Initial kernel scaffold; baseline (speedup 1.0000x reference)
import jax
import jax.numpy as jnp
from jax.experimental import pallas as pl


def kernel(x, adj, W, att_src, att_dst, bias):
    raise NotImplementedError("write your pallas kernel here")


# speedup vs baseline: 9662.7418x; 9662.7418x over previous
import functools

import jax
import jax.numpy as jnp
from jax.experimental import pallas as pl
from jax.experimental.pallas import tpu as pltpu

_NEG_SLOPE = 0.2


def _gat_kernel(x_ref, adj_ref, w_ref, as_ref, ad_ref, b_ref, out_ref, h_ref,
                *, block_i, n_nodes):
    i = pl.program_id(0)

    @pl.when(i == 0)
    def _():
        h_ref[...] = jnp.dot(x_ref[...], w_ref[...],
                             preferred_element_type=jnp.float32)

    h = h_ref[...]
    a_s = jax.lax.dot_general(h, as_ref[...],
                              (((1,), (1,)), ((), ())),
                              preferred_element_type=jnp.float32)
    h_blk = h_ref[pl.ds(i * block_i, block_i), :]
    a_d = jax.lax.dot_general(ad_ref[...], h_blk,
                              (((1,), (1,)), ((), ())),
                              preferred_element_type=jnp.float32)

    s_mat = a_s + a_d
    s_mat = jnp.where(s_mat > 0, s_mat, _NEG_SLOPE * s_mat)

    jj = jax.lax.broadcasted_iota(jnp.int32, (n_nodes, block_i), 0)
    ii = jax.lax.broadcasted_iota(jnp.int32, (n_nodes, block_i), 1) + i * block_i
    mask = (adj_ref[...] != 0) | (jj == ii)
    s_mat = jnp.where(mask, s_mat, -jnp.inf)

    m = jnp.max(s_mat, axis=0, keepdims=True)
    p = jnp.exp(s_mat - m)
    denom = jnp.sum(p, axis=0, keepdims=True)
    alpha = p / (denom + 1e-16)

    out = jax.lax.dot_general(alpha, h,
                              (((0,), (0,)), ((), ())),
                              preferred_element_type=jnp.float32)
    out_ref[...] = out + b_ref[...]


def kernel(x, adj, W, att_src, att_dst, bias):
    n, in_f = x.shape
    f = W.shape[1]
    att_s = att_src.reshape(1, f)
    att_d = att_dst.reshape(1, f)
    b = bias.reshape(1, f)

    block_i = 256
    grid = (n // block_i,)

    out = pl.pallas_call(
        functools.partial(_gat_kernel, block_i=block_i, n_nodes=n),
        grid=grid,
        in_specs=[
            pl.BlockSpec((n, in_f), lambda i: (0, 0)),
            pl.BlockSpec((n, block_i), lambda i: (0, i)),
            pl.BlockSpec((in_f, f), lambda i: (0, 0)),
            pl.BlockSpec((1, f), lambda i: (0, 0)),
            pl.BlockSpec((1, f), lambda i: (0, 0)),
            pl.BlockSpec((1, f), lambda i: (0, 0)),
        ],
        out_specs=pl.BlockSpec((block_i, f), lambda i: (i, 0)),
        out_shape=jax.ShapeDtypeStruct((n, f), jnp.float32),
        scratch_shapes=[pltpu.VMEM((n, f), jnp.float32)],
    )(x, adj, W, att_s, att_d, b)
    return out
